# baseline (device time: 40003 ns/iter reference)
import jax
import jax.numpy as jnp
from jax import lax
from jax.experimental import pallas as pl
from jax.experimental.pallas import tpu as pltpu

CHUNKS = [256, 224, 192, 128, 96, 64, 40, 24]
C = len(CHUNKS)
OFFS = [sum(CHUNKS[:i]) for i in range(C)]


def kernel(x):
    m, n = x.shape
    half = m // 2
    assert sum(CHUNKS) == half

    def body(
        x_hbm,
        out_hbm,
        xv,
        sv,
        rv,
        cp_sems,
        z_send_sems,
        z_recv_sems,
        x_send_sems,
        x_recv_sems,
        wbm_sems,
        wbo_sems,
    ):
        my_x = lax.axis_index("x")
        my_y = lax.axis_index("y")
        my_z = lax.axis_index("z")
        z_partner = (my_x, my_y, 1 - my_z)
        x_neighbor = (1 - my_x, my_y, my_z)

        my_base = my_x * half
        other_base = (1 - my_x) * half

        barrier_sem = pltpu.get_barrier_semaphore()
        for nbr in (z_partner, x_neighbor):
            pl.semaphore_signal(
                barrier_sem, inc=1,
                device_id=nbr, device_id_type=pl.DeviceIdType.MESH,
            )
        pl.semaphore_wait(barrier_sem, 2)

        cps = []
        for c in range(C):
            vrows = pl.ds(OFFS[c], CHUNKS[c])
            hrows = pl.ds(my_base + OFFS[c], CHUNKS[c])
            cp = pltpu.make_async_copy(x_hbm.at[hrows, :], xv.at[vrows, :], cp_sems.at[c])
            cp.start()
            cps.append(cp)

        z_rdmas = []
        for c in range(C):
            vrows = pl.ds(OFFS[c], CHUNKS[c])
            cps[c].wait()
            rdma = pltpu.make_async_remote_copy(
                src_ref=xv.at[vrows, :],
                dst_ref=sv.at[vrows, :],
                send_sem=z_send_sems.at[c],
                recv_sem=z_recv_sems.at[c],
                device_id=z_partner,
                device_id_type=pl.DeviceIdType.MESH,
            )
            rdma.start()
            z_rdmas.append(rdma)

        x_rdmas = []
        wbs = []
        for c in range(C):
            vrows = pl.ds(OFFS[c], CHUNKS[c])
            hrows = pl.ds(my_base + OFFS[c], CHUNKS[c])
            z_rdmas[c].wait_recv()
            sv[vrows, :] = sv[vrows, :] + xv[vrows, :]
            fwd = pltpu.make_async_remote_copy(
                src_ref=sv.at[vrows, :],
                dst_ref=rv.at[vrows, :],
                send_sem=x_send_sems.at[c],
                recv_sem=x_recv_sems.at[c],
                device_id=x_neighbor,
                device_id_type=pl.DeviceIdType.MESH,
            )
            fwd.start()
            x_rdmas.append(fwd)
            wb = pltpu.make_async_copy(sv.at[vrows, :], out_hbm.at[hrows, :], wbm_sems.at[c])
            wb.start()
            wbs.append(wb)

        for c in range(C):
            vrows = pl.ds(OFFS[c], CHUNKS[c])
            hrows = pl.ds(other_base + OFFS[c], CHUNKS[c])
            recv = pltpu.make_async_remote_copy(
                src_ref=rv.at[vrows, :],
                dst_ref=rv.at[vrows, :],
                send_sem=x_send_sems.at[c],
                recv_sem=x_recv_sems.at[c],
                device_id=x_neighbor,
                device_id_type=pl.DeviceIdType.MESH,
            )
            recv.wait_recv()
            wb = pltpu.make_async_copy(rv.at[vrows, :], out_hbm.at[hrows, :], wbo_sems.at[c])
            wb.start()
            wbs.append(wb)

        for wb in wbs:
            wb.wait()
        for c in range(C):
            z_rdmas[c].wait_send()
            x_rdmas[c].wait_send()

    return pl.pallas_call(
        body,
        out_shape=jax.ShapeDtypeStruct((m, n), x.dtype),
        in_specs=[pl.BlockSpec(memory_space=pl.ANY)],
        out_specs=pl.BlockSpec(memory_space=pl.ANY),
        scratch_shapes=[
            pltpu.VMEM((half, n), x.dtype),
            pltpu.VMEM((half, n), x.dtype),
            pltpu.VMEM((half, n), x.dtype),
            pltpu.SemaphoreType.DMA((C,)),
            pltpu.SemaphoreType.DMA((C,)),
            pltpu.SemaphoreType.DMA((C,)),
            pltpu.SemaphoreType.DMA((C,)),
            pltpu.SemaphoreType.DMA((C,)),
            pltpu.SemaphoreType.DMA((C,)),
            pltpu.SemaphoreType.DMA((C,)),
        ],
        compiler_params=pltpu.CompilerParams(collective_id=0),
    )(x)
